# baseline (speedup 1.0000x reference)
import functools

import jax
import jax.numpy as jnp
from jax.experimental import pallas as pl
from jax.experimental.pallas import tpu as pltpu


def _body(x_ref, W1_ref, b1_ref, W2_ref, b2_ref, E_ref, Et_ref,
          Wd1_ref, bd1_ref, Wd2_ref, bd2_ref, out_ref):
    x = x_ref[...]
    h = jnp.maximum(
        jnp.dot(x, W1_ref[...], preferred_element_type=jnp.float32)
        + b1_ref[...], 0.0)
    z = jnp.maximum(
        jnp.dot(h, W2_ref[...], preferred_element_type=jnp.float32)
        + b2_ref[...], 0.0)
    E = E_ref[...]
    sim = jnp.dot(z, E, preferred_element_type=jnp.float32)
    z_sq = jnp.sum(z * z, axis=1, keepdims=True)
    e_sq = jnp.sum(E * E, axis=0, keepdims=True)
    dist = z_sq + e_sq - 2.0 * sim
    idx = jnp.argmin(dist, axis=1)
    k_iota = jax.lax.broadcasted_iota(jnp.int32, dist.shape, 1)
    onehot = (k_iota == idx[:, None]).astype(jnp.bfloat16)
    quant = jnp.dot(onehot, Et_ref[...],
                    preferred_element_type=jnp.float32)
    q = z + (quant - z)
    hd = jnp.maximum(
        jnp.dot(q, Wd1_ref[...], preferred_element_type=jnp.float32)
        + bd1_ref[...], 0.0)
    out_ref[...] = (
        jnp.dot(hd, Wd2_ref[...], preferred_element_type=jnp.float32)
        + bd2_ref[...])


@jax.jit
def kernel(x, W1, b1, W2, b2, E, Wd1, bd1, Wd2, bd2):
    B, D = x.shape
    L, K = E.shape
    Dh = W1.shape[1]
    TB = min(512, B)
    grid = (B // TB,)

    def batch_map(i):
        return (i, 0)

    def const_map(i):
        return (0, 0)

    full = lambda shape: pl.BlockSpec(shape, const_map)
    out = pl.pallas_call(
        _body,
        grid=grid,
        in_specs=[
            pl.BlockSpec((TB, D), batch_map),
            full((D, Dh)),
            full((1, Dh)),
            full((Dh, L)),
            full((1, L)),
            full((L, K)),
            pl.BlockSpec((K, L), const_map),
            full((L, Dh)),
            full((1, Dh)),
            full((Dh, D)),
            full((1, D)),
        ],
        out_specs=pl.BlockSpec((TB, D), batch_map),
        out_shape=jax.ShapeDtypeStruct((B, D), jnp.float32),
        compiler_params=pltpu.CompilerParams(
            dimension_semantics=("arbitrary",),
        ),
    )(x, W1, b1.reshape(1, -1), W2, b2.reshape(1, -1), E,
      E.T.astype(jnp.bfloat16),
      Wd1, bd1.reshape(1, -1), Wd2, bd2.reshape(1, -1))
    return out


# speedup vs baseline: 1.2068x; 1.2068x over previous
import functools

import jax
import jax.numpy as jnp
from jax.experimental import pallas as pl
from jax.experimental.pallas import tpu as pltpu


def _body(x_ref, W1_ref, b1_ref, W2_ref, b2_ref, E_ref, Et_ref,
          Wd1_ref, bd1_ref, Wd2_ref, bd2_ref, out_ref):
    x = x_ref[...]
    h = jnp.maximum(
        jnp.dot(x, W1_ref[...], preferred_element_type=jnp.float32)
        + b1_ref[...], 0.0)
    z = jnp.maximum(
        jnp.dot(h, W2_ref[...], preferred_element_type=jnp.float32)
        + b2_ref[...], 0.0)
    E = E_ref[...]
    sim = jnp.dot(z, E, preferred_element_type=jnp.float32)
    z_sq = jnp.sum(z * z, axis=1, keepdims=True)
    e_sq = jnp.sum(E * E, axis=0, keepdims=True)
    dist = z_sq + e_sq - 2.0 * sim
    idx = jnp.argmin(dist, axis=1)
    k_iota = jax.lax.broadcasted_iota(jnp.int32, dist.shape, 1)
    onehot = (k_iota == idx[:, None]).astype(jnp.float32)
    quant = jnp.dot(onehot, Et_ref[...], preferred_element_type=jnp.float32)
    q = z + (quant - z)
    hd = jnp.maximum(
        jnp.dot(q, Wd1_ref[...], preferred_element_type=jnp.float32)
        + bd1_ref[...], 0.0)
    out_ref[...] = (
        jnp.dot(hd, Wd2_ref[...], preferred_element_type=jnp.float32)
        + bd2_ref[...])


@jax.jit
def kernel(x, W1, b1, W2, b2, E, Wd1, bd1, Wd2, bd2):
    B, D = x.shape
    L, K = E.shape
    Dh = W1.shape[1]
    TB = min(1024, B)
    grid = (B // TB,)

    def batch_map(i):
        return (i, 0)

    def const_map(i):
        return (0, 0)

    full = lambda shape: pl.BlockSpec(shape, const_map)
    out = pl.pallas_call(
        _body,
        grid=grid,
        in_specs=[
            pl.BlockSpec((TB, D), batch_map),
            full((D, Dh)),
            full((1, Dh)),
            full((Dh, L)),
            full((1, L)),
            full((L, K)),
            pl.BlockSpec((K, L), const_map),
            full((L, Dh)),
            full((1, Dh)),
            full((Dh, D)),
            full((1, D)),
        ],
        out_specs=pl.BlockSpec((TB, D), batch_map),
        out_shape=jax.ShapeDtypeStruct((B, D), jnp.float32),
        compiler_params=pltpu.CompilerParams(
            dimension_semantics=("arbitrary",),
        ),
    )(x, W1, b1.reshape(1, -1), W2, b2.reshape(1, -1), E,
      E.T,
      Wd1, bd1.reshape(1, -1), Wd2, bd2.reshape(1, -1))
    return out


# speedup vs baseline: 1.2193x; 1.0103x over previous
import functools

import jax
import jax.numpy as jnp
from jax.experimental import pallas as pl
from jax.experimental.pallas import tpu as pltpu


def _body(x_ref, W1_ref, b1_ref, W2_ref, b2_ref, E_ref, Et_ref,
          Wd1_ref, bd1_ref, Wd2_ref, bd2_ref, out_ref):
    x = x_ref[...]
    h = jnp.maximum(
        jnp.dot(x, W1_ref[...], preferred_element_type=jnp.float32)
        + b1_ref[...], 0.0)
    z = jnp.maximum(
        jnp.dot(h, W2_ref[...], preferred_element_type=jnp.float32)
        + b2_ref[...], 0.0)
    E = E_ref[...]
    sim = jnp.dot(z, E, preferred_element_type=jnp.float32)
    z_sq = jnp.sum(z * z, axis=1, keepdims=True)
    e_sq = jnp.sum(E * E, axis=0, keepdims=True)
    dist = z_sq + e_sq - 2.0 * sim
    idx = jnp.argmin(dist, axis=1)
    k_iota = jax.lax.broadcasted_iota(jnp.int32, dist.shape, 1)
    onehot = (k_iota == idx[:, None]).astype(jnp.float32)
    quant = jnp.dot(onehot, Et_ref[...], preferred_element_type=jnp.float32)
    q = z + (quant - z)
    hd = jnp.maximum(
        jnp.dot(q, Wd1_ref[...], preferred_element_type=jnp.float32)
        + bd1_ref[...], 0.0)
    out_ref[...] = (
        jnp.dot(hd, Wd2_ref[...], preferred_element_type=jnp.float32)
        + bd2_ref[...])


@jax.jit
def kernel(x, W1, b1, W2, b2, E, Wd1, bd1, Wd2, bd2):
    B, D = x.shape
    L, K = E.shape
    Dh = W1.shape[1]
    TB = min(2048, B)
    grid = (B // TB,)

    def batch_map(i):
        return (i, 0)

    def const_map(i):
        return (0, 0)

    full = lambda shape: pl.BlockSpec(shape, const_map)
    out = pl.pallas_call(
        _body,
        grid=grid,
        in_specs=[
            pl.BlockSpec((TB, D), batch_map),
            full((D, Dh)),
            full((1, Dh)),
            full((Dh, L)),
            full((1, L)),
            full((L, K)),
            pl.BlockSpec((K, L), const_map),
            full((L, Dh)),
            full((1, Dh)),
            full((Dh, D)),
            full((1, D)),
        ],
        out_specs=pl.BlockSpec((TB, D), batch_map),
        out_shape=jax.ShapeDtypeStruct((B, D), jnp.float32),
        compiler_params=pltpu.CompilerParams(
            dimension_semantics=("arbitrary",),
        ),
    )(x, W1, b1.reshape(1, -1), W2, b2.reshape(1, -1), E,
      E.T,
      Wd1, bd1.reshape(1, -1), Wd2, bd2.reshape(1, -1))
    return out
